# baseline (speedup 1.0000x reference)
import functools

import jax
import jax.numpy as jnp
from jax import lax
from jax.experimental import pallas as pl
from jax.experimental.pallas import tpu as pltpu
from jax.experimental.pallas import tpu_sc as plsc

B = 16
N = 2048
L = 16
H = N // 2
NCB = N // L
NQB = H // L
RG = 4
BIG = 3.0e38


def _chamfer_body(q_hbm, k_hbm, out1, out2,
                  qx_v, qy_v, qz_v, kx_v, ky_v, kz_v,
                  na_v, nb_v, rmin_v, cmin_v):
    c = lax.axis_index("c")
    s = lax.axis_index("s")
    wid = s * 2 + c
    b = wid // 2
    h = wid % 2

    pltpu.sync_copy(q_hbm.at[b, h, 0], qx_v)
    pltpu.sync_copy(q_hbm.at[b, h, 1], qy_v)
    pltpu.sync_copy(q_hbm.at[b, h, 2], qz_v)
    pltpu.sync_copy(k_hbm.at[b, 0], kx_v)
    pltpu.sync_copy(k_hbm.at[b, 1], ky_v)
    pltpu.sync_copy(k_hbm.at[b, 2], kz_v)

    big = jnp.full((L,), BIG, jnp.float32)
    lanes = lax.iota(jnp.int32, L)

    def bf16_round(x):
        u = lax.bitcast_convert_type(x, jnp.int32)
        u = u + 0x7FFF + ((u >> 16) & 1)
        u = u & jnp.int32(-65536)
        return lax.bitcast_convert_type(u, jnp.float32)

    def prep_cols(j, _):
        off = j * L
        x = kx_v[pl.ds(off, L)]
        y = ky_v[pl.ds(off, L)]
        z = kz_v[pl.ds(off, L)]
        nb_v[pl.ds(off, L)] = x * x + y * y + z * z
        kx_v[pl.ds(off, L)] = bf16_round(x)
        ky_v[pl.ds(off, L)] = bf16_round(y)
        kz_v[pl.ds(off, L)] = bf16_round(z)
        cmin_v[pl.ds(off, L)] = big
        return 0

    lax.fori_loop(0, NCB, prep_cols, 0, unroll=2)

    def prep_rows(j, _):
        off = j * L
        x = qx_v[pl.ds(off, L)]
        y = qy_v[pl.ds(off, L)]
        z = qz_v[pl.ds(off, L)]
        na_v[pl.ds(off, L)] = x * x + y * y + z * z
        qx_v[pl.ds(off, L)] = -2.0 * bf16_round(x)
        qy_v[pl.ds(off, L)] = -2.0 * bf16_round(y)
        qz_v[pl.ds(off, L)] = -2.0 * bf16_round(z)
        return 0

    lax.fori_loop(0, NQB, prep_rows, 0, unroll=2)

    def macro_group(g, _):
        mbase = g * L
        acc = big
        for sub in range(L // RG):
            rows = []
            for r in range(RG):
                idx = jnp.full((L,), mbase + sub * RG + r, jnp.int32)
                m0 = plsc.load_gather(qx_v, [idx])
                m1 = plsc.load_gather(qy_v, [idx])
                m2 = plsc.load_gather(qz_v, [idx])
                na = plsc.load_gather(na_v, [idx])
                rows.append((m0, m1, m2, na))

            def col_block(j, carry):
                rm = list(carry)
                off = j * L
                bx = kx_v[pl.ds(off, L)]
                by = ky_v[pl.ds(off, L)]
                bz = kz_v[pl.ds(off, L)]
                nb = nb_v[pl.ds(off, L)]
                cm = cmin_v[pl.ds(off, L)]
                for r in range(RG):
                    m0, m1, m2, na = rows[r]
                    d = (na + nb) + m0 * bx + m1 * by + m2 * bz
                    rm[r] = jnp.minimum(rm[r], d)
                    cm = jnp.minimum(cm, d)
                cmin_v[pl.ds(off, L)] = cm
                return tuple(rm)

            rmins = lax.fori_loop(0, NCB, col_block,
                                  tuple(big for _ in range(RG)), unroll=2)
            for r in range(RG):
                smin = jnp.min(rmins[r])
                acc = jnp.where(lanes == (sub * RG + r), smin, acc)
        rmin_v[pl.ds(mbase, L)] = acc
        return 0

    lax.fori_loop(0, H // L, macro_group, 0)

    pltpu.sync_copy(rmin_v, out1.at[b, h])
    pltpu.sync_copy(cmin_v, out2.at[b, h])


@functools.partial(
    pl.kernel,
    mesh=plsc.VectorSubcoreMesh(core_axis_name="c", subcore_axis_name="s"),
    compiler_params=pltpu.CompilerParams(
        use_tc_tiling_on_sc=False, needs_layout_passes=False),
    out_type=[
        jax.ShapeDtypeStruct((B, 2, H), jnp.float32),
        jax.ShapeDtypeStruct((B, 2, N), jnp.float32),
    ],
    scratch_types=[
        pltpu.VMEM((H,), jnp.float32),
        pltpu.VMEM((H,), jnp.float32),
        pltpu.VMEM((H,), jnp.float32),
        pltpu.VMEM((N,), jnp.float32),
        pltpu.VMEM((N,), jnp.float32),
        pltpu.VMEM((N,), jnp.float32),
        pltpu.VMEM((H,), jnp.float32),
        pltpu.VMEM((N,), jnp.float32),
        pltpu.VMEM((H,), jnp.float32),
        pltpu.VMEM((N,), jnp.float32),
    ],
)
def _chamfer_sc(q_hbm, k_hbm, out1, out2,
                qx_v, qy_v, qz_v, kx_v, ky_v, kz_v,
                na_v, nb_v, rmin_v, cmin_v):
    _chamfer_body(q_hbm, k_hbm, out1, out2,
                  qx_v, qy_v, qz_v, kx_v, ky_v, kz_v,
                  na_v, nb_v, rmin_v, cmin_v)


TR = 256
NT = N // TR


def _chamfer_tc_body(a_ref, b_ref, out1_ref, out2_ref):
    i = pl.program_id(1)
    d = jax.lax.dot_general(
        a_ref[0], b_ref[0],
        dimension_numbers=(((1,), (1,)), ((), ())),
        preferred_element_type=jnp.float32,
    )
    out1_ref[0, 0, pl.ds(i * TR, TR)] = jnp.min(d, axis=1)
    cmin = jnp.min(d, axis=0, keepdims=True)

    @pl.when(i == 0)
    def _():
        out2_ref[0, 0] = cmin[0]

    @pl.when(i > 0)
    def _():
        out2_ref[0, 0] = jnp.minimum(out2_ref[0, 0], cmin[0])


def _chamfer_tc(a7, b7, nb_batches):
    return pl.pallas_call(
        _chamfer_tc_body,
        grid=(nb_batches, NT),
        in_specs=[
            pl.BlockSpec((1, TR, 8), lambda b, i: (b, i, 0)),
            pl.BlockSpec((1, N, 8), lambda b, i: (b, 0, 0)),
        ],
        out_specs=[
            pl.BlockSpec((1, 1, N), lambda b, i: (b, 0, 0)),
            pl.BlockSpec((1, 1, N), lambda b, i: (b, 0, 0)),
        ],
        out_shape=[
            jax.ShapeDtypeStruct((nb_batches, 1, N), jnp.float32),
            jax.ShapeDtypeStruct((nb_batches, 1, N), jnp.float32),
        ],
        compiler_params=pltpu.CompilerParams(
            dimension_semantics=("parallel", "arbitrary"),
        ),
    )(a7, b7)


def _mxu_operands(xyz1, xyz2):
    x1b = xyz1.astype(jnp.bfloat16)
    x2b = xyz2.astype(jnp.bfloat16)
    na = jnp.sum(xyz1 * xyz1, axis=-1)
    nb = jnp.sum(xyz2 * xyz2, axis=-1)
    nah = na.astype(jnp.bfloat16)
    nal = (na - nah.astype(jnp.float32)).astype(jnp.bfloat16)
    nbh = nb.astype(jnp.bfloat16)
    nbl = (nb - nbh.astype(jnp.float32)).astype(jnp.bfloat16)
    one = jnp.ones_like(nah)
    zero = jnp.zeros_like(nah)
    a7 = jnp.stack([-2.0 * x1b[..., 0], -2.0 * x1b[..., 1], -2.0 * x1b[..., 2],
                    nah, nal, one, one, zero], axis=-1)
    b7 = jnp.stack([x2b[..., 0], x2b[..., 1], x2b[..., 2],
                    one, one, nbh, nbl, zero], axis=-1)
    return a7, b7


@jax.jit
def kernel(xyz1, xyz2):
    a7, b7 = _mxu_operands(xyz1, xyz2)
    d1sq, d2sq = _chamfer_tc(a7, b7, B)
    d1sq = d1sq[:, 0]
    d2sq = d2sq[:, 0]
    d1 = jnp.sqrt(jnp.maximum(d1sq, 1e-12))
    d2 = jnp.sqrt(jnp.maximum(d2sq, 1e-12))
    return 0.5 * (jnp.mean(d1) + jnp.mean(d2))


@jax.jit
def _kernel_sc_only(xyz1, xyz2):
    q = xyz1.reshape(B, 2, H, 3).transpose(0, 1, 3, 2)
    k = xyz2.transpose(0, 2, 1)
    rminsq, cminsq_part = _chamfer_sc(q, k)
    d1sq = rminsq.reshape(B, N)
    d2sq = jnp.min(cminsq_part, axis=1)
    d1 = jnp.sqrt(jnp.maximum(d1sq, 1e-12))
    d2 = jnp.sqrt(jnp.maximum(d2sq, 1e-12))
    return 0.5 * (jnp.mean(d1) + jnp.mean(d2))


# speedup vs baseline: 2.2105x; 2.2105x over previous
import functools

import jax
import jax.numpy as jnp
from jax import lax
from jax.experimental import pallas as pl
from jax.experimental.pallas import tpu as pltpu
from jax.experimental.pallas import tpu_sc as plsc

B = 16
N = 2048
L = 16
H = N // 2
NCB = N // L
NQB = H // L
RG = 4
BIG = 3.0e38


def _chamfer_body(q_hbm, k_hbm, out1, out2,
                  qx_v, qy_v, qz_v, kx_v, ky_v, kz_v,
                  na_v, nb_v, rmin_v, cmin_v):
    c = lax.axis_index("c")
    s = lax.axis_index("s")
    wid = s * 2 + c
    b = wid // 2
    h = wid % 2

    pltpu.sync_copy(q_hbm.at[b, h, 0], qx_v)
    pltpu.sync_copy(q_hbm.at[b, h, 1], qy_v)
    pltpu.sync_copy(q_hbm.at[b, h, 2], qz_v)
    pltpu.sync_copy(k_hbm.at[b, 0], kx_v)
    pltpu.sync_copy(k_hbm.at[b, 1], ky_v)
    pltpu.sync_copy(k_hbm.at[b, 2], kz_v)

    big = jnp.full((L,), BIG, jnp.float32)
    lanes = lax.iota(jnp.int32, L)

    def bf16_round(x):
        u = lax.bitcast_convert_type(x, jnp.int32)
        u = u + 0x7FFF + ((u >> 16) & 1)
        u = u & jnp.int32(-65536)
        return lax.bitcast_convert_type(u, jnp.float32)

    def prep_cols(j, _):
        off = j * L
        x = kx_v[pl.ds(off, L)]
        y = ky_v[pl.ds(off, L)]
        z = kz_v[pl.ds(off, L)]
        nb_v[pl.ds(off, L)] = x * x + y * y + z * z
        kx_v[pl.ds(off, L)] = bf16_round(x)
        ky_v[pl.ds(off, L)] = bf16_round(y)
        kz_v[pl.ds(off, L)] = bf16_round(z)
        cmin_v[pl.ds(off, L)] = big
        return 0

    lax.fori_loop(0, NCB, prep_cols, 0, unroll=2)

    def prep_rows(j, _):
        off = j * L
        x = qx_v[pl.ds(off, L)]
        y = qy_v[pl.ds(off, L)]
        z = qz_v[pl.ds(off, L)]
        na_v[pl.ds(off, L)] = x * x + y * y + z * z
        qx_v[pl.ds(off, L)] = -2.0 * bf16_round(x)
        qy_v[pl.ds(off, L)] = -2.0 * bf16_round(y)
        qz_v[pl.ds(off, L)] = -2.0 * bf16_round(z)
        return 0

    lax.fori_loop(0, NQB, prep_rows, 0, unroll=2)

    def macro_group(g, _):
        mbase = g * L
        acc = big
        for sub in range(L // RG):
            rows = []
            for r in range(RG):
                idx = jnp.full((L,), mbase + sub * RG + r, jnp.int32)
                m0 = plsc.load_gather(qx_v, [idx])
                m1 = plsc.load_gather(qy_v, [idx])
                m2 = plsc.load_gather(qz_v, [idx])
                na = plsc.load_gather(na_v, [idx])
                rows.append((m0, m1, m2, na))

            def col_block(j, carry):
                rm = list(carry)
                off = j * L
                bx = kx_v[pl.ds(off, L)]
                by = ky_v[pl.ds(off, L)]
                bz = kz_v[pl.ds(off, L)]
                nb = nb_v[pl.ds(off, L)]
                cm = cmin_v[pl.ds(off, L)]
                for r in range(RG):
                    m0, m1, m2, na = rows[r]
                    d = (na + nb) + m0 * bx + m1 * by + m2 * bz
                    rm[r] = jnp.minimum(rm[r], d)
                    cm = jnp.minimum(cm, d)
                cmin_v[pl.ds(off, L)] = cm
                return tuple(rm)

            rmins = lax.fori_loop(0, NCB, col_block,
                                  tuple(big for _ in range(RG)), unroll=2)
            for r in range(RG):
                smin = jnp.min(rmins[r])
                acc = jnp.where(lanes == (sub * RG + r), smin, acc)
        rmin_v[pl.ds(mbase, L)] = acc
        return 0

    lax.fori_loop(0, H // L, macro_group, 0)

    pltpu.sync_copy(rmin_v, out1.at[b, h])
    pltpu.sync_copy(cmin_v, out2.at[b, h])


@functools.partial(
    pl.kernel,
    mesh=plsc.VectorSubcoreMesh(core_axis_name="c", subcore_axis_name="s"),
    compiler_params=pltpu.CompilerParams(
        use_tc_tiling_on_sc=False, needs_layout_passes=False),
    out_type=[
        jax.ShapeDtypeStruct((B, 2, H), jnp.float32),
        jax.ShapeDtypeStruct((B, 2, N), jnp.float32),
    ],
    scratch_types=[
        pltpu.VMEM((H,), jnp.float32),
        pltpu.VMEM((H,), jnp.float32),
        pltpu.VMEM((H,), jnp.float32),
        pltpu.VMEM((N,), jnp.float32),
        pltpu.VMEM((N,), jnp.float32),
        pltpu.VMEM((N,), jnp.float32),
        pltpu.VMEM((H,), jnp.float32),
        pltpu.VMEM((N,), jnp.float32),
        pltpu.VMEM((H,), jnp.float32),
        pltpu.VMEM((N,), jnp.float32),
    ],
)
def _chamfer_sc(q_hbm, k_hbm, out1, out2,
                qx_v, qy_v, qz_v, kx_v, ky_v, kz_v,
                na_v, nb_v, rmin_v, cmin_v):
    _chamfer_body(q_hbm, k_hbm, out1, out2,
                  qx_v, qy_v, qz_v, kx_v, ky_v, kz_v,
                  na_v, nb_v, rmin_v, cmin_v)


TR = 256
NT = N // TR


def _bf16_hilo(v):
    hi = v.astype(jnp.bfloat16)
    lo = (v - hi.astype(jnp.float32)).astype(jnp.bfloat16)
    return hi, lo


def _chamfer_tc_body(x1_ref, x2_ref, out1_ref, out2_ref, b8_ref):
    i = pl.program_id(1)

    @pl.when(i == 0)
    def _():
        x2 = x2_ref[0]
        nb = jnp.sum(x2 * x2, axis=1, keepdims=True)
        nbh, nbl = _bf16_hilo(nb)
        one = jnp.ones_like(nbh)
        b8_ref[...] = jnp.concatenate(
            [x2.astype(jnp.bfloat16), one, one, nbh, nbl], axis=1)

    x1 = x1_ref[0]
    na = jnp.sum(x1 * x1, axis=1, keepdims=True)
    nah, nal = _bf16_hilo(na)
    one = jnp.ones_like(nah)
    a8 = jnp.concatenate(
        [(-2.0 * x1).astype(jnp.bfloat16), nah, nal, one, one], axis=1)
    d = jax.lax.dot_general(
        a8, b8_ref[...],
        dimension_numbers=(((1,), (1,)), ((), ())),
        preferred_element_type=jnp.float32,
    )
    out1_ref[0, 0, pl.ds(i * TR, TR)] = jnp.min(d, axis=1)
    cmin = jnp.min(d, axis=0, keepdims=True)

    @pl.when(i == 0)
    def _():
        out2_ref[0, 0] = cmin[0]

    @pl.when(i > 0)
    def _():
        out2_ref[0, 0] = jnp.minimum(out2_ref[0, 0], cmin[0])


def _chamfer_tc(xyz1, xyz2, nb_batches):
    return pl.pallas_call(
        _chamfer_tc_body,
        grid=(nb_batches, NT),
        in_specs=[
            pl.BlockSpec((1, TR, 3), lambda b, i: (b, i, 0)),
            pl.BlockSpec((1, N, 3), lambda b, i: (b, 0, 0)),
        ],
        out_specs=[
            pl.BlockSpec((1, 1, N), lambda b, i: (b, 0, 0)),
            pl.BlockSpec((1, 1, N), lambda b, i: (b, 0, 0)),
        ],
        out_shape=[
            jax.ShapeDtypeStruct((nb_batches, 1, N), jnp.float32),
            jax.ShapeDtypeStruct((nb_batches, 1, N), jnp.float32),
        ],
        scratch_shapes=[pltpu.VMEM((N, 7), jnp.bfloat16)],
        compiler_params=pltpu.CompilerParams(
            dimension_semantics=("parallel", "arbitrary"),
        ),
    )(xyz1, xyz2)


@jax.jit
def kernel(xyz1, xyz2):
    d1sq, d2sq = _chamfer_tc(xyz1, xyz2, B)
    d1sq = d1sq[:, 0]
    d2sq = d2sq[:, 0]
    d1 = jnp.sqrt(jnp.maximum(d1sq, 1e-12))
    d2 = jnp.sqrt(jnp.maximum(d2sq, 1e-12))
    return 0.5 * (jnp.mean(d1) + jnp.mean(d2))


@jax.jit
def _kernel_sc_only(xyz1, xyz2):
    q = xyz1.reshape(B, 2, H, 3).transpose(0, 1, 3, 2)
    k = xyz2.transpose(0, 2, 1)
    rminsq, cminsq_part = _chamfer_sc(q, k)
    d1sq = rminsq.reshape(B, N)
    d2sq = jnp.min(cminsq_part, axis=1)
    d1 = jnp.sqrt(jnp.maximum(d1sq, 1e-12))
    d2 = jnp.sqrt(jnp.maximum(d2sq, 1e-12))
    return 0.5 * (jnp.mean(d1) + jnp.mean(d2))


# speedup vs baseline: 2.7100x; 1.2260x over previous
import functools

import jax
import jax.numpy as jnp
from jax import lax
from jax.experimental import pallas as pl
from jax.experimental.pallas import tpu as pltpu
from jax.experimental.pallas import tpu_sc as plsc

B = 16
N = 2048
L = 16
H = N // 2
NCB = N // L
NQB = H // L
RG = 4
BIG = 3.0e38


def _chamfer_body(q_hbm, k_hbm, out1, out2,
                  qx_v, qy_v, qz_v, kx_v, ky_v, kz_v,
                  na_v, nb_v, rmin_v, cmin_v):
    c = lax.axis_index("c")
    s = lax.axis_index("s")
    wid = s * 2 + c
    b = wid // 2
    h = wid % 2

    pltpu.sync_copy(q_hbm.at[b, h, 0], qx_v)
    pltpu.sync_copy(q_hbm.at[b, h, 1], qy_v)
    pltpu.sync_copy(q_hbm.at[b, h, 2], qz_v)
    pltpu.sync_copy(k_hbm.at[b, 0], kx_v)
    pltpu.sync_copy(k_hbm.at[b, 1], ky_v)
    pltpu.sync_copy(k_hbm.at[b, 2], kz_v)

    big = jnp.full((L,), BIG, jnp.float32)
    lanes = lax.iota(jnp.int32, L)

    def bf16_round(x):
        u = lax.bitcast_convert_type(x, jnp.int32)
        u = u + 0x7FFF + ((u >> 16) & 1)
        u = u & jnp.int32(-65536)
        return lax.bitcast_convert_type(u, jnp.float32)

    def prep_cols(j, _):
        off = j * L
        x = kx_v[pl.ds(off, L)]
        y = ky_v[pl.ds(off, L)]
        z = kz_v[pl.ds(off, L)]
        nb_v[pl.ds(off, L)] = x * x + y * y + z * z
        kx_v[pl.ds(off, L)] = bf16_round(x)
        ky_v[pl.ds(off, L)] = bf16_round(y)
        kz_v[pl.ds(off, L)] = bf16_round(z)
        cmin_v[pl.ds(off, L)] = big
        return 0

    lax.fori_loop(0, NCB, prep_cols, 0, unroll=2)

    def prep_rows(j, _):
        off = j * L
        x = qx_v[pl.ds(off, L)]
        y = qy_v[pl.ds(off, L)]
        z = qz_v[pl.ds(off, L)]
        na_v[pl.ds(off, L)] = x * x + y * y + z * z
        qx_v[pl.ds(off, L)] = -2.0 * bf16_round(x)
        qy_v[pl.ds(off, L)] = -2.0 * bf16_round(y)
        qz_v[pl.ds(off, L)] = -2.0 * bf16_round(z)
        return 0

    lax.fori_loop(0, NQB, prep_rows, 0, unroll=2)

    def macro_group(g, _):
        mbase = g * L
        acc = big
        for sub in range(L // RG):
            rows = []
            for r in range(RG):
                idx = jnp.full((L,), mbase + sub * RG + r, jnp.int32)
                m0 = plsc.load_gather(qx_v, [idx])
                m1 = plsc.load_gather(qy_v, [idx])
                m2 = plsc.load_gather(qz_v, [idx])
                na = plsc.load_gather(na_v, [idx])
                rows.append((m0, m1, m2, na))

            def col_block(j, carry):
                rm = list(carry)
                off = j * L
                bx = kx_v[pl.ds(off, L)]
                by = ky_v[pl.ds(off, L)]
                bz = kz_v[pl.ds(off, L)]
                nb = nb_v[pl.ds(off, L)]
                cm = cmin_v[pl.ds(off, L)]
                for r in range(RG):
                    m0, m1, m2, na = rows[r]
                    d = (na + nb) + m0 * bx + m1 * by + m2 * bz
                    rm[r] = jnp.minimum(rm[r], d)
                    cm = jnp.minimum(cm, d)
                cmin_v[pl.ds(off, L)] = cm
                return tuple(rm)

            rmins = lax.fori_loop(0, NCB, col_block,
                                  tuple(big for _ in range(RG)), unroll=2)
            for r in range(RG):
                smin = jnp.min(rmins[r])
                acc = jnp.where(lanes == (sub * RG + r), smin, acc)
        rmin_v[pl.ds(mbase, L)] = acc
        return 0

    lax.fori_loop(0, H // L, macro_group, 0)

    pltpu.sync_copy(rmin_v, out1.at[b, h])
    pltpu.sync_copy(cmin_v, out2.at[b, h])


@functools.partial(
    pl.kernel,
    mesh=plsc.VectorSubcoreMesh(core_axis_name="c", subcore_axis_name="s"),
    compiler_params=pltpu.CompilerParams(
        use_tc_tiling_on_sc=False, needs_layout_passes=False),
    out_type=[
        jax.ShapeDtypeStruct((B, 2, H), jnp.float32),
        jax.ShapeDtypeStruct((B, 2, N), jnp.float32),
    ],
    scratch_types=[
        pltpu.VMEM((H,), jnp.float32),
        pltpu.VMEM((H,), jnp.float32),
        pltpu.VMEM((H,), jnp.float32),
        pltpu.VMEM((N,), jnp.float32),
        pltpu.VMEM((N,), jnp.float32),
        pltpu.VMEM((N,), jnp.float32),
        pltpu.VMEM((H,), jnp.float32),
        pltpu.VMEM((N,), jnp.float32),
        pltpu.VMEM((H,), jnp.float32),
        pltpu.VMEM((N,), jnp.float32),
    ],
)
def _chamfer_sc(q_hbm, k_hbm, out1, out2,
                qx_v, qy_v, qz_v, kx_v, ky_v, kz_v,
                na_v, nb_v, rmin_v, cmin_v):
    _chamfer_body(q_hbm, k_hbm, out1, out2,
                  qx_v, qy_v, qz_v, kx_v, ky_v, kz_v,
                  na_v, nb_v, rmin_v, cmin_v)


TR = 512
NT = N // TR


def _bf16_hilo(v):
    hi = v.astype(jnp.bfloat16)
    lo = (v - hi.astype(jnp.float32)).astype(jnp.bfloat16)
    return hi, lo


def _chamfer_tc_body(x1_ref, x2_ref, out1_ref, out2_ref, b8_ref):
    i = pl.program_id(1)

    @pl.when(i == 0)
    def _():
        x2 = x2_ref[0]
        nb = jnp.sum(x2 * x2, axis=1, keepdims=True)
        nbh, nbl = _bf16_hilo(nb)
        one = jnp.ones_like(nbh)
        b8_ref[...] = jnp.transpose(jnp.concatenate(
            [x2.astype(jnp.bfloat16), one, one, nbh, nbl], axis=1))

    x1 = x1_ref[0]
    na = jnp.sum(x1 * x1, axis=1, keepdims=True)
    nah, nal = _bf16_hilo(na)
    one = jnp.ones_like(nah)
    a8 = jnp.concatenate(
        [(-2.0 * x1).astype(jnp.bfloat16), nah, nal, one, one], axis=1)
    d = jax.lax.dot_general(
        a8, b8_ref[...],
        dimension_numbers=(((1,), (0,)), ((), ())),
        preferred_element_type=jnp.float32,
    )
    out1_ref[0, 0, pl.ds(i * TR, TR)] = jnp.min(d, axis=1)
    cmin = jnp.min(d, axis=0, keepdims=True)

    @pl.when(i == 0)
    def _():
        out2_ref[0, 0] = cmin[0]

    @pl.when(i > 0)
    def _():
        out2_ref[0, 0] = jnp.minimum(out2_ref[0, 0], cmin[0])


def _chamfer_tc(xyz1, xyz2, nb_batches):
    return pl.pallas_call(
        _chamfer_tc_body,
        grid=(nb_batches, NT),
        in_specs=[
            pl.BlockSpec((1, TR, 3), lambda b, i: (b, i, 0)),
            pl.BlockSpec((1, N, 3), lambda b, i: (b, 0, 0)),
        ],
        out_specs=[
            pl.BlockSpec((1, 1, N), lambda b, i: (b, 0, 0)),
            pl.BlockSpec((1, 1, N), lambda b, i: (b, 0, 0)),
        ],
        out_shape=[
            jax.ShapeDtypeStruct((nb_batches, 1, N), jnp.float32),
            jax.ShapeDtypeStruct((nb_batches, 1, N), jnp.float32),
        ],
        scratch_shapes=[pltpu.VMEM((7, N), jnp.bfloat16)],
        compiler_params=pltpu.CompilerParams(
            dimension_semantics=("parallel", "arbitrary"),
        ),
    )(xyz1, xyz2)


@jax.jit
def kernel(xyz1, xyz2):
    d1sq, d2sq = _chamfer_tc(xyz1, xyz2, B)
    d1sq = d1sq[:, 0]
    d2sq = d2sq[:, 0]
    d1 = jnp.sqrt(jnp.maximum(d1sq, 1e-12))
    d2 = jnp.sqrt(jnp.maximum(d2sq, 1e-12))
    return 0.5 * (jnp.mean(d1) + jnp.mean(d2))


@jax.jit
def _kernel_sc_only(xyz1, xyz2):
    q = xyz1.reshape(B, 2, H, 3).transpose(0, 1, 3, 2)
    k = xyz2.transpose(0, 2, 1)
    rminsq, cminsq_part = _chamfer_sc(q, k)
    d1sq = rminsq.reshape(B, N)
    d2sq = jnp.min(cminsq_part, axis=1)
    d1 = jnp.sqrt(jnp.maximum(d1sq, 1e-12))
    d2 = jnp.sqrt(jnp.maximum(d2sq, 1e-12))
    return 0.5 * (jnp.mean(d1) + jnp.mean(d2))
